# baseline (speedup 1.0000x reference)
import jax
import jax.numpy as jnp
from jax import lax
from jax.experimental import pallas as pl
from jax.experimental.pallas import tpu as pltpu

B = 1024
N_PATCHES = 128
D_MODEL = 768
BB = 16
NBUF = 4
NSTEP = B // BB


def _body(rp_ref, w_ref, s_ref, out_hbm, buf, sems):
    i = pl.program_id(0)
    slot = lax.rem(i, NBUF)
    wv = (w_ref[...] * s_ref[0, 0]).reshape(1, 1, D_MODEL)

    @pl.when(i >= NBUF)
    def _reclaim():
        pltpu.make_async_copy(
            buf.at[slot], out_hbm.at[pl.ds(i * BB, BB), :, :], sems.at[slot]
        ).wait()

    buf[slot] = rp_ref[pl.ds(i * BB, BB), :][:, :, None] * wv
    pltpu.make_async_copy(
        buf.at[slot], out_hbm.at[pl.ds(i * BB, BB), :, :], sems.at[slot]
    ).start()

    @pl.when(i == NSTEP - 1)
    def _drain():
        for k in range(NBUF):
            pltpu.make_async_copy(
                buf.at[k], out_hbm.at[pl.ds(0, BB), :, :], sems.at[k]
            ).wait()


def kernel(n_patches, relative_positions, W, scale):
    w2 = W.reshape(1, D_MODEL)
    s2 = scale.reshape(1, 1)
    out = pl.pallas_call(
        _body,
        grid=(NSTEP,),
        in_specs=[
            pl.BlockSpec((B, N_PATCHES), lambda i: (0, 0)),
            pl.BlockSpec((1, D_MODEL), lambda i: (0, 0)),
            pl.BlockSpec((1, 1), lambda i: (0, 0)),
        ],
        out_specs=pl.BlockSpec(memory_space=pl.ANY),
        out_shape=jax.ShapeDtypeStruct((B, N_PATCHES, D_MODEL), jnp.float32),
        scratch_shapes=[
            pltpu.VMEM((NBUF, BB, N_PATCHES, D_MODEL), jnp.float32),
            pltpu.SemaphoreType.DMA((NBUF,)),
        ],
    )(relative_positions, w2, s2)
    return out
